# baseline (speedup 1.0000x reference)
import functools

import jax
import jax.numpy as jnp
from jax import lax
from jax.experimental import pallas as pl
from jax.experimental.pallas import tpu as pltpu
from jax.experimental.pallas import tpu_sc as plsc

EMBED_D = 64
BATCH_N = 16384

_NC = 2
_NS = 16
_NW = _NC * _NS
_B_PER_W = BATCH_N // _NW
_HALF = _B_PER_W // 2


def _make_gather():
    mesh = plsc.VectorSubcoreMesh(core_axis_name="c", subcore_axis_name="s")

    @functools.partial(
        pl.kernel,
        mesh=mesh,
        out_type=jax.ShapeDtypeStruct((_NW, _B_PER_W, EMBED_D), jnp.float32),
        scratch_types=[
            pltpu.VMEM((_B_PER_W,), jnp.int32),
            pltpu.VMEM((_HALF, EMBED_D), jnp.float32),
            pltpu.VMEM_SHARED((_NS, _HALF, EMBED_D), jnp.float32),
            pltpu.SemaphoreType.DMA,
            pltpu.SemaphoreType.DMA,
            pltpu.SemaphoreType.DMA,
        ],
        compiler_params=pltpu.CompilerParams(
            use_tc_tiling_on_sc=True, needs_layout_passes=False
        ),
    )
    def k(idx_hbm, table_hbm, out_hbm, idx_v, rows_v, rows_s, sem_a, sem_b,
          sem_out):
        sid = lax.axis_index("s")
        wid = sid * _NC + lax.axis_index("c")
        pltpu.sync_copy(idx_hbm.at[wid], idx_v)

        def body_a(t, carry):
            base = t * 16
            ev = idx_v[pl.ds(base, 16)]
            for l in range(16):
                pltpu.async_copy(
                    table_hbm.at[ev[l]], rows_v.at[base + l], sem_a
                )
            return carry

        def body_b(t, carry):
            base = _HALF + t * 16
            ev = idx_v[pl.ds(base, 16)]
            for l in range(16):
                pltpu.async_copy(
                    table_hbm.at[ev[l]],
                    rows_s.at[sid, base - _HALF + l],
                    sem_b,
                )
            return carry

        lax.fori_loop(0, _HALF // 16, body_a, 0)
        lax.fori_loop(0, _HALF // 16, body_b, 0)
        pltpu.make_async_copy(
            out_hbm.at[wid, pl.ds(0, _HALF)], rows_v, sem_a
        ).wait()
        pltpu.make_async_copy(
            out_hbm.at[wid, pl.ds(_HALF, _HALF)], rows_s.at[sid], sem_b
        ).wait()
        pltpu.async_copy(rows_v, out_hbm.at[wid, pl.ds(0, _HALF)], sem_out)
        pltpu.async_copy(
            rows_s.at[sid], out_hbm.at[wid, pl.ds(_HALF, _HALF)], sem_out
        ).wait()
        pltpu.make_async_copy(
            out_hbm.at[wid, pl.ds(0, _HALF)], rows_v, sem_out
        ).wait()

    return k


_gather = _make_gather()


def kernel(entities, table):
    idx = entities.astype(jnp.int32).reshape(_NW, _B_PER_W)
    out = _gather(idx, table)
    return out.reshape(BATCH_N, EMBED_D)
